# speedup vs baseline: 1.0265x; 1.0265x over previous
import functools
import math

import jax
import jax.numpy as jnp
from jax import lax
from jax.experimental import pallas as pl
from jax.experimental.pallas import tpu as pltpu
from jax.experimental.pallas import tpu_sc as plsc

D = 1024
B = 32768
SCALE = math.sqrt(D)

NC = 2
NS = 16
NW = NC * NS
B_PER_W = B // NW
CHUNK = 64
LANES = 16

_mesh = plsc.VectorSubcoreMesh(core_axis_name="c", subcore_axis_name="s")


@functools.partial(
    pl.kernel,
    mesh=_mesh,
    out_type=jax.ShapeDtypeStruct((B, D), jnp.float32),
    scratch_types=[
        pltpu.VMEM((B_PER_W,), jnp.int32),
        pltpu.VMEM((CHUNK, D), jnp.float32),
        pltpu.SemaphoreType.DMA,
    ],
)
def _gather_scale(table_hbm, idx_hbm, out_hbm, idx_v, rows_v, sem):
    wid = lax.axis_index("s") * NC + lax.axis_index("c")
    base = wid * B_PER_W
    pltpu.sync_copy(idx_hbm.at[pl.ds(base, B_PER_W)], idx_v)

    @pl.loop(0, B_PER_W, step=CHUNK)
    def _step(j):
        pltpu.async_copy(table_hbm.at[idx_v.at[pl.ds(j, CHUNK)]], rows_v, sem).wait()

        @pl.loop(0, CHUNK)
        def _row(r):
            for c in range(0, D, LANES):
                rows_v[r, pl.ds(c, LANES)] = rows_v[r, pl.ds(c, LANES)] * SCALE

        pltpu.sync_copy(rows_v, out_hbm.at[pl.ds(base + j, CHUNK)])


def kernel(x, table):
    idx = x.reshape(B).astype(jnp.int32)
    out = _gather_scale(table, idx)
    return out.reshape(x.shape[0], x.shape[1], D)


# speedup vs baseline: 1.5095x; 1.4705x over previous
import functools
import math

import jax
import jax.numpy as jnp
from jax import lax
from jax.experimental import pallas as pl
from jax.experimental.pallas import tpu as pltpu
from jax.experimental.pallas import tpu_sc as plsc

D = 1024
B = 32768
SCALE = math.sqrt(D)

NC = 2
NS = 16
NW = NC * NS
B_PER_W = B // NW
CHUNK = 32
LANES = 16

_mesh = plsc.VectorSubcoreMesh(core_axis_name="c", subcore_axis_name="s")


@functools.partial(
    pl.kernel,
    mesh=_mesh,
    out_type=jax.ShapeDtypeStruct((B, D), jnp.float32),
    scratch_types=[
        pltpu.VMEM((B_PER_W,), jnp.int32),
        pltpu.VMEM((2, CHUNK, D), jnp.float32),
        pltpu.SemaphoreType.DMA,
        pltpu.SemaphoreType.DMA,
        pltpu.SemaphoreType.DMA,
        pltpu.SemaphoreType.DMA,
    ],
)
def _gather_scale(table_hbm, idx_hbm, out_hbm, idx_v, rows_v, g0, g1, s0, s1):
    wid = lax.axis_index("s") * NC + lax.axis_index("c")
    base = wid * B_PER_W
    pltpu.sync_copy(idx_hbm.at[pl.ds(base, B_PER_W)], idx_v)

    gsem = (g0, g1)
    ssem = (s0, s1)

    def gather(buf, j):
        return pltpu.make_async_copy(
            table_hbm.at[idx_v.at[pl.ds(j, CHUNK)]], rows_v.at[buf], gsem[buf])

    def store(buf, j):
        return pltpu.make_async_copy(
            rows_v.at[buf], out_hbm.at[pl.ds(base + j, CHUNK)], ssem[buf])

    def scale(buf):
        @pl.loop(0, CHUNK)
        def _row(r):
            for c in range(0, D, LANES):
                rows_v[buf, r, pl.ds(c, LANES)] = (
                    rows_v[buf, r, pl.ds(c, LANES)] * SCALE)

    gather(0, 0).start()
    gather(1, CHUNK).start()

    @pl.loop(0, B_PER_W, step=2 * CHUNK)
    def _step(j):
        for buf in range(2):
            jj = j + buf * CHUNK
            gather(buf, jj).wait()
            scale(buf)
            store(buf, jj).start()

        @pl.when(j + 2 * CHUNK < B_PER_W)
        def _prefetch():
            for buf in range(2):
                jj = j + buf * CHUNK
                store(buf, jj).wait()
                gather(buf, jj + 2 * CHUNK).start()

    store(0, B_PER_W - 2 * CHUNK).wait()
    store(1, B_PER_W - CHUNK).wait()


def kernel(x, table):
    idx = x.reshape(B).astype(jnp.int32)
    out = _gather_scale(table, idx)
    return out.reshape(x.shape[0], x.shape[1], D)


# speedup vs baseline: 1.6162x; 1.0707x over previous
import functools
import math

import jax
import jax.numpy as jnp
from jax import lax
from jax.experimental import pallas as pl
from jax.experimental.pallas import tpu as pltpu
from jax.experimental.pallas import tpu_sc as plsc

D = 1024
B = 32768
SCALE = math.sqrt(D)

NC = 2
NS = 16
NW = NC * NS
B_PER_W = B // NW
CHUNK = 16
NBUF = 4
NSTEP = B_PER_W // CHUNK
LANES = 16

_mesh = plsc.VectorSubcoreMesh(core_axis_name="c", subcore_axis_name="s")


@functools.partial(
    pl.kernel,
    mesh=_mesh,
    out_type=jax.ShapeDtypeStruct((B, D), jnp.float32),
    scratch_types=[
        pltpu.VMEM((B_PER_W,), jnp.int32),
        pltpu.VMEM((NBUF, CHUNK, D), jnp.float32),
    ]
    + [pltpu.SemaphoreType.DMA] * (2 * NBUF),
)
def _gather_scale(table_hbm, idx_hbm, out_hbm, idx_v, rows_v, *sems):
    gsem = sems[:NBUF]
    ssem = sems[NBUF:]
    wid = lax.axis_index("s") * NC + lax.axis_index("c")
    base = wid * B_PER_W
    pltpu.sync_copy(idx_hbm.at[pl.ds(base, B_PER_W)], idx_v)

    def gather(buf, g):
        return pltpu.make_async_copy(
            table_hbm.at[idx_v.at[pl.ds(g * CHUNK, CHUNK)]],
            rows_v.at[buf], gsem[buf])

    def store(buf, g):
        return pltpu.make_async_copy(
            rows_v.at[buf], out_hbm.at[pl.ds(base + g * CHUNK, CHUNK)],
            ssem[buf])

    def scale(buf):
        @pl.loop(0, CHUNK)
        def _row(r):
            for c in range(0, D, LANES):
                rows_v[buf, r, pl.ds(c, LANES)] = (
                    rows_v[buf, r, pl.ds(c, LANES)] * SCALE)

    gather(0, 0).start()
    gather(1, 1).start()

    @pl.loop(0, NSTEP, step=NBUF)
    def _step(j):
        for s in range(NBUF):
            buf = s
            g = j + s
            gather(buf, g).wait()
            scale(buf)
            store(buf, g).start()

            pbuf = (s + 2) % NBUF

            @pl.when(g + 2 < NSTEP)
            def _prefetch():
                @pl.when(g >= 2)
                def _drain():
                    store(pbuf, g - 2).wait()

                gather(pbuf, g + 2).start()

    for s in range(NBUF):
        store(s, NSTEP - NBUF + s).wait()


def kernel(x, table):
    idx = x.reshape(B).astype(jnp.int32)
    out = _gather_scale(table, idx)
    return out.reshape(x.shape[0], x.shape[1], D)


# speedup vs baseline: 1.7144x; 1.0608x over previous
import functools
import math

import jax
import jax.numpy as jnp
from jax import lax
from jax.experimental import pallas as pl
from jax.experimental.pallas import tpu as pltpu
from jax.experimental.pallas import tpu_sc as plsc

D = 1024
B = 32768
SCALE = math.sqrt(D)

NC = 2
NS = 16
NW = NC * NS
B_PER_W = B // NW
CHUNK = 16
NBUF = 4
NSTEP = B_PER_W // CHUNK
LANES = 16

_mesh = plsc.VectorSubcoreMesh(core_axis_name="c", subcore_axis_name="s")


@functools.partial(
    pl.kernel,
    mesh=_mesh,
    out_type=jax.ShapeDtypeStruct((B, D), jnp.float32),
    scratch_types=[
        pltpu.VMEM((B_PER_W,), jnp.int32),
        pltpu.VMEM((NBUF, CHUNK, D), jnp.float32),
    ]
    + [pltpu.SemaphoreType.DMA] * (2 * NBUF),
)
def _gather_scale(table_hbm, idx_hbm, out_hbm, idx_v, rows_v, *sems):
    gsem = sems[:NBUF]
    ssem = sems[NBUF:]
    wid = lax.axis_index("s") * NC + lax.axis_index("c")
    base = wid * B_PER_W
    pltpu.sync_copy(idx_hbm.at[pl.ds(base, B_PER_W)], idx_v)

    def gather(buf, g):
        return pltpu.make_async_copy(
            table_hbm.at[idx_v.at[pl.ds(g * CHUNK, CHUNK)]],
            rows_v.at[buf], gsem[buf])

    def store(buf, g):
        return pltpu.make_async_copy(
            rows_v.at[buf], out_hbm.at[pl.ds(base + g * CHUNK, CHUNK)],
            ssem[buf])

    def scale(buf):
        @pl.loop(0, CHUNK, unroll=2)
        def _row(r):
            for c in range(0, D, LANES):
                rows_v[buf, r, pl.ds(c, LANES)] = (
                    rows_v[buf, r, pl.ds(c, LANES)] * SCALE)

    gather(0, 0).start()
    gather(1, 1).start()

    @pl.loop(0, NSTEP, step=NBUF)
    def _step(j):
        for s in range(NBUF):
            buf = s
            g = j + s
            gather(buf, g).wait()

            pbuf = (s + 2) % NBUF

            @pl.when(g + 2 < NSTEP)
            def _prefetch():
                @pl.when(g >= 2)
                def _drain():
                    store(pbuf, g - 2).wait()

                gather(pbuf, g + 2).start()

            scale(buf)
            store(buf, g).start()

    for s in range(NBUF):
        store(s, NSTEP - NBUF + s).wait()


def kernel(x, table):
    idx = x.reshape(B).astype(jnp.int32)
    out = _gather_scale(table, idx)
    return out.reshape(x.shape[0], x.shape[1], D)
